# baseline (speedup 1.0000x reference)
import functools

import jax
import jax.numpy as jnp
from jax import lax
from jax.experimental import pallas as pl
from jax.experimental.pallas import tpu as pltpu

E = 8
K = 2
D = 768
H = 3072
T = 4096
BAL_COEF = 0.01
Z_COEF = 0.001

BT = 256
NBT = T // BT
B = 256
P = K * T + E * B
NB = P // B
HT = 4
HB = H // HT

_LANES = 128


def _router_body(x_ref, wg_ref, i0_ref, i1_ref, g0_ref, g1_ref, r0_ref,
                 r1_ref, counts_ref, aux_ref, run_ref, me_ref, z_ref):
    b = pl.program_id(0)

    @pl.when(b == 0)
    def _init():
        run_ref[...] = jnp.zeros_like(run_ref)
        me_ref[...] = jnp.zeros_like(me_ref)
        z_ref[...] = jnp.zeros_like(z_ref)

    xb = x_ref[...]
    logits = jnp.dot(xb, wg_ref[...], preferred_element_type=jnp.float32)
    lane = lax.broadcasted_iota(jnp.int32, (BT, _LANES), 1)
    lm = jnp.where(lane < E, logits, -1e30)
    mx = jnp.max(lm, axis=1, keepdims=True)
    ex = jnp.exp(lm - mx)
    se = jnp.sum(ex, axis=1, keepdims=True)
    lse = jnp.log(se) + mx
    probs = ex / se

    m1 = jnp.max(probs, axis=1, keepdims=True)
    i1 = jnp.min(jnp.where(probs == m1, lane, _LANES), axis=1, keepdims=True)
    pm = jnp.where(lane == i1, -1.0, probs)
    m2 = jnp.max(pm, axis=1, keepdims=True)
    i2 = jnp.min(jnp.where((pm == m2) & (lane < E), lane, _LANES), axis=1,
                 keepdims=True)
    s = m1 + m2
    g0 = m1 / s
    g1 = m2 / s

    oh = ((lane == i1) | (lane == i2)).astype(jnp.float32)
    row = lax.broadcasted_iota(jnp.int32, (BT, BT), 0)
    col = lax.broadcasted_iota(jnp.int32, (BT, BT), 1)
    tril = (row > col).astype(jnp.float32)
    within = jnp.dot(tril, oh, preferred_element_type=jnp.float32)
    rank = within + run_ref[...]

    i0_ref[...] = jnp.min(jnp.where(probs == m1, lane, _LANES), axis=1)
    i1_ref[...] = jnp.min(jnp.where((pm == m2) & (lane < E), lane, _LANES),
                          axis=1)
    g0_ref[...] = g0[:, 0]
    g1_ref[...] = g1[:, 0]
    r0_ref[...] = jnp.sum(jnp.where(lane == i1, rank, 0.0), axis=1).astype(
        jnp.int32)
    r1_ref[...] = jnp.sum(jnp.where(lane == i2, rank, 0.0), axis=1).astype(
        jnp.int32)

    run_ref[...] += jnp.sum(oh, axis=0, keepdims=True)
    me_ref[...] += jnp.sum(probs, axis=0, keepdims=True)
    z_ref[...] += jnp.sum(lse * lse)[None, None]

    @pl.when(b == NBT - 1)
    def _fin():
        counts = run_ref[...]
        z_loss = Z_COEF * z_ref[0, 0] / T
        bal = BAL_COEF * E * jnp.sum((me_ref[...] / T) * (counts / T))
        aux_ref[...] = (z_loss + bal)[None, None]
        counts_ref[...] = counts.astype(jnp.int32)


def _run_router(x, wg_pad):
    out_shapes = (
        jax.ShapeDtypeStruct((T,), jnp.int32),
        jax.ShapeDtypeStruct((T,), jnp.int32),
        jax.ShapeDtypeStruct((T,), jnp.float32),
        jax.ShapeDtypeStruct((T,), jnp.float32),
        jax.ShapeDtypeStruct((T,), jnp.int32),
        jax.ShapeDtypeStruct((T,), jnp.int32),
        jax.ShapeDtypeStruct((1, _LANES), jnp.int32),
        jax.ShapeDtypeStruct((1, 1), jnp.float32),
    )
    vec_spec = pl.BlockSpec((BT,), lambda b: (b,))
    return pl.pallas_call(
        _router_body,
        grid=(NBT,),
        in_specs=[
            pl.BlockSpec((BT, D), lambda b: (b, 0)),
            pl.BlockSpec((D, _LANES), lambda b: (0, 0)),
        ],
        out_specs=(
            vec_spec, vec_spec, vec_spec, vec_spec, vec_spec, vec_spec,
            pl.BlockSpec((1, _LANES), lambda b: (0, 0)),
            pl.BlockSpec((1, 1), lambda b: (0, 0)),
        ),
        out_shape=out_shapes,
        scratch_shapes=[
            pltpu.VMEM((1, _LANES), jnp.float32),
            pltpu.VMEM((1, _LANES), jnp.float32),
            pltpu.VMEM((1, 1), jnp.float32),
        ],
        compiler_params=pltpu.CompilerParams(
            dimension_semantics=("arbitrary",)),
    )(x, wg_pad)


def _ffn_body(bm_ref, x_ref, w1_ref, b1_ref, w2_ref, b2_ref, y_ref):
    xb = x_ref[...].astype(jnp.bfloat16)
    w1 = w1_ref[0].astype(jnp.bfloat16)
    h = jnp.dot(xb, w1, preferred_element_type=jnp.float32)
    h = jax.nn.gelu(h + b1_ref[0, 0][None, :])
    w2 = w2_ref[0].astype(jnp.bfloat16)
    y_ref[...] = (jnp.dot(h.astype(jnp.bfloat16), w2,
                          preferred_element_type=jnp.float32)
                  + b2_ref[0, 0][None, :])


def _run_ffn(blk_expert, x_sorted, W1, b1, W2, b2):
    grid_spec = pltpu.PrefetchScalarGridSpec(
        num_scalar_prefetch=1,
        grid=(NB,),
        in_specs=[
            pl.BlockSpec((B, D), lambda b, bm: (b, 0)),
            pl.BlockSpec((1, D, H), lambda b, bm: (bm[b], 0, 0)),
            pl.BlockSpec((1, 1, H), lambda b, bm: (bm[b], 0, 0)),
            pl.BlockSpec((1, H, D), lambda b, bm: (bm[b], 0, 0)),
            pl.BlockSpec((1, 1, D), lambda b, bm: (bm[b], 0, 0)),
        ],
        out_specs=pl.BlockSpec((B, D), lambda b, bm: (b, 0)),
    )
    return pl.pallas_call(
        _ffn_body,
        grid_spec=grid_spec,
        out_shape=jax.ShapeDtypeStruct((P, D), jnp.float32),
        compiler_params=pltpu.CompilerParams(
            dimension_semantics=("arbitrary",)),
    )(blk_expert, x_sorted, W1, b1.reshape(E, 1, H),
      W2, b2.reshape(E, 1, D))


def kernel(x, Wg, W1, b1, W2, b2):
    wg_pad = jnp.zeros((D, _LANES), jnp.float32).at[:, :E].set(Wg)
    topi0, topi1, g0, g1, r0, r1, counts_o, aux = _run_router(x, wg_pad)

    counts = counts_o[0, :E]
    pcblk = (counts + B - 1) // B
    off_e = jnp.concatenate(
        [jnp.zeros((1,), jnp.int32), jnp.cumsum(pcblk * B)])[:E]
    boundaries = jnp.cumsum(pcblk)
    bidx = jnp.arange(NB, dtype=jnp.int32)
    e_of_b = jnp.sum((bidx[:, None] >= boundaries[None, :]).astype(jnp.int32),
                     axis=1)
    blk_expert = jnp.minimum(e_of_b, E - 1).astype(jnp.int32)

    pos0 = off_e[topi0] + r0
    pos1 = off_e[topi1] + r1
    tok = jnp.arange(T, dtype=jnp.int32)
    perm = jnp.zeros((P,), jnp.int32).at[pos0].set(tok).at[pos1].set(tok)
    x_sorted = x[perm]

    y_sorted = _run_ffn(blk_expert, x_sorted, W1, b1, W2, b2)

    out = g0[:, None] * y_sorted[pos0] + g1[:, None] * y_sorted[pos1]

    return out.astype(x.dtype), aux[0, 0]


# speedup vs baseline: 1.0162x; 1.0162x over previous
import functools

import jax
import jax.numpy as jnp
from jax import lax
from jax.experimental import pallas as pl
from jax.experimental.pallas import tpu as pltpu

E = 8
K = 2
D = 768
H = 3072
T = 4096
BAL_COEF = 0.01
Z_COEF = 0.001

BT = 256
NBT = T // BT
B = 256
P = K * T + E * B
NB = P // B
HT = 4
HB = H // HT

_LANES = 128


def _router_body(x_ref, wg_ref, i0_ref, i1_ref, g0_ref, g1_ref, r0_ref,
                 r1_ref, counts_ref, aux_ref, run_ref, me_ref, z_ref):
    b = pl.program_id(0)

    @pl.when(b == 0)
    def _init():
        run_ref[...] = jnp.zeros_like(run_ref)
        me_ref[...] = jnp.zeros_like(me_ref)
        z_ref[...] = jnp.zeros_like(z_ref)

    xb = x_ref[...]
    logits = jnp.dot(xb, wg_ref[...], preferred_element_type=jnp.float32)
    lane = lax.broadcasted_iota(jnp.int32, (BT, _LANES), 1)
    lm = jnp.where(lane < E, logits, -1e30)
    mx = jnp.max(lm, axis=1, keepdims=True)
    ex = jnp.exp(lm - mx)
    se = jnp.sum(ex, axis=1, keepdims=True)
    lse = jnp.log(se) + mx
    probs = ex / se

    m1 = jnp.max(probs, axis=1, keepdims=True)
    i1 = jnp.min(jnp.where(probs == m1, lane, _LANES), axis=1, keepdims=True)
    pm = jnp.where(lane == i1, -1.0, probs)
    m2 = jnp.max(pm, axis=1, keepdims=True)
    i2 = jnp.min(jnp.where((pm == m2) & (lane < E), lane, _LANES), axis=1,
                 keepdims=True)
    s = m1 + m2
    g0 = m1 / s
    g1 = m2 / s

    oh = ((lane == i1) | (lane == i2)).astype(jnp.float32)
    row = lax.broadcasted_iota(jnp.int32, (BT, BT), 0)
    col = lax.broadcasted_iota(jnp.int32, (BT, BT), 1)
    tril = (row > col).astype(jnp.float32)
    within = jnp.dot(tril, oh, preferred_element_type=jnp.float32)
    rank = within + run_ref[...]

    i0_ref[...] = jnp.min(jnp.where(probs == m1, lane, _LANES), axis=1)
    i1_ref[...] = jnp.min(jnp.where((pm == m2) & (lane < E), lane, _LANES),
                          axis=1)
    g0_ref[...] = g0[:, 0]
    g1_ref[...] = g1[:, 0]
    r0_ref[...] = jnp.sum(jnp.where(lane == i1, rank, 0.0), axis=1).astype(
        jnp.int32)
    r1_ref[...] = jnp.sum(jnp.where(lane == i2, rank, 0.0), axis=1).astype(
        jnp.int32)

    run_ref[...] += jnp.sum(oh, axis=0, keepdims=True)
    me_ref[...] += jnp.sum(probs, axis=0, keepdims=True)
    z_ref[...] += jnp.sum(lse * lse)[None, None]

    @pl.when(b == NBT - 1)
    def _fin():
        counts = run_ref[...]
        z_loss = Z_COEF * z_ref[0, 0] / T
        bal = BAL_COEF * E * jnp.sum((me_ref[...] / T) * (counts / T))
        aux_ref[...] = (z_loss + bal)[None, None]
        counts_ref[...] = counts.astype(jnp.int32)


def _run_router(x, wg_pad):
    out_shapes = (
        jax.ShapeDtypeStruct((T,), jnp.int32),
        jax.ShapeDtypeStruct((T,), jnp.int32),
        jax.ShapeDtypeStruct((T,), jnp.float32),
        jax.ShapeDtypeStruct((T,), jnp.float32),
        jax.ShapeDtypeStruct((T,), jnp.int32),
        jax.ShapeDtypeStruct((T,), jnp.int32),
        jax.ShapeDtypeStruct((1, _LANES), jnp.int32),
        jax.ShapeDtypeStruct((1, 1), jnp.float32),
    )
    vec_spec = pl.BlockSpec((BT,), lambda b: (b,))
    return pl.pallas_call(
        _router_body,
        grid=(NBT,),
        in_specs=[
            pl.BlockSpec((BT, D), lambda b: (b, 0)),
            pl.BlockSpec((D, _LANES), lambda b: (0, 0)),
        ],
        out_specs=(
            vec_spec, vec_spec, vec_spec, vec_spec, vec_spec, vec_spec,
            pl.BlockSpec((1, _LANES), lambda b: (0, 0)),
            pl.BlockSpec((1, 1), lambda b: (0, 0)),
        ),
        out_shape=out_shapes,
        scratch_shapes=[
            pltpu.VMEM((1, _LANES), jnp.float32),
            pltpu.VMEM((1, _LANES), jnp.float32),
            pltpu.VMEM((1, 1), jnp.float32),
        ],
        compiler_params=pltpu.CompilerParams(
            dimension_semantics=("arbitrary",)),
    )(x, wg_pad)


def _ffn_body(bm_ref, x_ref, w1_ref, b1_ref, w2_ref, b2_ref, y_ref):
    h = jnp.dot(x_ref[...], w1_ref[0], preferred_element_type=jnp.float32)
    h = jax.nn.gelu(h + b1_ref[0, 0][None, :])
    y_ref[...] = (jnp.dot(h, w2_ref[0], preferred_element_type=jnp.float32)
                  + b2_ref[0, 0][None, :])


def _run_ffn(blk_expert, x_sorted, W1, b1, W2, b2):
    grid_spec = pltpu.PrefetchScalarGridSpec(
        num_scalar_prefetch=1,
        grid=(NB,),
        in_specs=[
            pl.BlockSpec((B, D), lambda b, bm: (b, 0)),
            pl.BlockSpec((1, D, H), lambda b, bm: (bm[b], 0, 0)),
            pl.BlockSpec((1, 1, H), lambda b, bm: (bm[b], 0, 0)),
            pl.BlockSpec((1, H, D), lambda b, bm: (bm[b], 0, 0)),
            pl.BlockSpec((1, 1, D), lambda b, bm: (bm[b], 0, 0)),
        ],
        out_specs=pl.BlockSpec((B, D), lambda b, bm: (b, 0)),
    )
    return pl.pallas_call(
        _ffn_body,
        grid_spec=grid_spec,
        out_shape=jax.ShapeDtypeStruct((P, D), jnp.float32),
        compiler_params=pltpu.CompilerParams(
            dimension_semantics=("arbitrary",)),
    )(blk_expert, x_sorted, W1, b1.reshape(E, 1, H),
      W2, b2.reshape(E, 1, D))


def kernel(x, Wg, W1, b1, W2, b2):
    wg_pad = jnp.zeros((D, _LANES), jnp.float32).at[:, :E].set(Wg)
    topi0, topi1, g0, g1, r0, r1, counts_o, aux = _run_router(x, wg_pad)

    counts = counts_o[0, :E]
    pcblk = (counts + B - 1) // B
    off_e = jnp.concatenate(
        [jnp.zeros((1,), jnp.int32), jnp.cumsum(pcblk * B)])[:E]
    boundaries = jnp.cumsum(pcblk)
    bidx = jnp.arange(NB, dtype=jnp.int32)
    e_of_b = jnp.sum((bidx[:, None] >= boundaries[None, :]).astype(jnp.int32),
                     axis=1)
    blk_expert = jnp.minimum(e_of_b, E - 1).astype(jnp.int32)

    pos0 = off_e[topi0] + r0
    pos1 = off_e[topi1] + r1
    tok = jnp.arange(T, dtype=jnp.int32)
    perm = (jnp.zeros((P,), jnp.int32)
            .at[pos0].set(tok, unique_indices=True, mode="promise_in_bounds")
            .at[pos1].set(tok, unique_indices=True, mode="promise_in_bounds"))
    x_sorted = x.at[perm].get(mode="promise_in_bounds")

    y_sorted = _run_ffn(blk_expert, x_sorted, W1, b1, W2, b2)

    y0 = y_sorted.at[pos0].get(unique_indices=True, mode="promise_in_bounds")
    y1 = y_sorted.at[pos1].get(unique_indices=True, mode="promise_in_bounds")
    out = g0[:, None] * y0 + g1[:, None] * y1

    return out.astype(x.dtype), aux[0, 0]
